# baseline (speedup 1.0000x reference)
import functools

import jax
import jax.numpy as jnp
from jax import lax
from jax.experimental import pallas as pl
from jax.experimental.pallas import tpu as pltpu
from jax.experimental.pallas import tpu_sc as plsc

B = 64
T = 64
DRGB = 2048
DFLOW = 2048
EMB = 1024
H = 1024
CD = 128
MOM = 0.999

_F32 = jnp.float32
_BF16 = jnp.bfloat16


def _sc_mom_kernel(whh_ref, whhk_ref, out_ref, a_v, b_v):
    ns = lax.axis_size("s")
    nc = lax.axis_size("c")
    wid = lax.axis_index("c") * ns + lax.axis_index("s")
    rows_per = H // (nc * ns)
    base = wid * rows_per
    nvec = (3 * H) // 16

    def _chunk(ci, carry):
        r0 = base + ci * 8
        pltpu.sync_copy(whh_ref.at[pl.ds(r0, 8), :], a_v)
        pltpu.sync_copy(whhk_ref.at[pl.ds(r0, 8), :], b_v)

        def _row(r, c2):
            def _vec(i, c3):
                sl = pl.ds(i * 16, 16)
                b_v[r, sl] = MOM * b_v[r, sl] + (1.0 - MOM) * a_v[r, sl]
                return c3
            return lax.fori_loop(0, nvec, _vec, c2)

        lax.fori_loop(0, 8, _row, carry)
        pltpu.sync_copy(b_v, out_ref.at[pl.ds(r0, 8), :])
        return carry

    lax.fori_loop(0, rows_per // 8, _chunk, 0)


def _mm1_kernel(rgb_ref, flow_ref, w1_ref, w1k_ref, b1_ref, b1k_ref,
                g1_ref, be1_ref, g1k_ref, be1k_ref,
                wih_ref, wihk_ref,
                bih_ref, bihk_ref, bhh_ref, bhhk_ref,
                xq_ref, xk_ref, oih_ref, obih_ref, obhh_ref,
                *, rows_per_blk):
    oih_ref[...] = (MOM * wihk_ref[...]
                    + (1.0 - MOM) * wih_ref[...]).astype(_BF16)

    @pl.when(pl.program_id(0) == 0)
    def _biases():
        obih_ref[...] = MOM * bihk_ref[...] + (1.0 - MOM) * bih_ref[...]
        obhh_ref[...] = MOM * bhhk_ref[...] + (1.0 - MOM) * bhh_ref[...]

    rgb = rgb_ref[...]
    flow = flow_ref[...]
    yq = (jnp.dot(rgb, w1_ref[0:DRGB, :], preferred_element_type=_F32)
          + jnp.dot(flow, w1_ref[DRGB:, :], preferred_element_type=_F32)
          + b1_ref[...])
    ykraw = (jnp.dot(rgb, w1k_ref[0:DRGB, :], preferred_element_type=_F32)
             + jnp.dot(flow, w1k_ref[DRGB:, :], preferred_element_type=_F32)
             + b1k_ref[...])
    yk = MOM * ykraw + (1.0 - MOM) * yq
    nb = rows_per_blk // T

    def _ln_relu(y, g, be):
        mu = jnp.mean(y, axis=-1, keepdims=True)
        var = jnp.mean(y * y, axis=-1, keepdims=True) - mu * mu
        a = jax.lax.rsqrt(var + 1e-5) * g
        out = y * a + (be - mu * a)
        out = jnp.maximum(out, 0.0).astype(_BF16)
        return out.reshape(nb, T, EMB).transpose(1, 0, 2)

    g1 = g1_ref[...]
    be1 = be1_ref[...]
    g1k = MOM * g1k_ref[...] + (1.0 - MOM) * g1
    be1k = MOM * be1k_ref[...] + (1.0 - MOM) * be1
    xq_ref[...] = _ln_relu(yq, g1, be1)
    xk_ref[...] = _ln_relu(yk, g1k, be1k)


def _mm2_kernel(xq_ref, xk_ref, wih_ref, wihk_ref, bih_ref, bihk_ref,
                giq_ref, gik_ref, *, t_blk):
    bih = bih_ref[...]
    bihk = bihk_ref[...]
    xq = xq_ref[...].reshape(t_blk * B, EMB)
    xk = xk_ref[...].reshape(t_blk * B, EMB)
    giq = jnp.dot(xq, wih_ref[...], preferred_element_type=_F32) + bih
    gik = jnp.dot(xk, wihk_ref[...], preferred_element_type=_F32) + bihk
    giq_ref[...] = giq.reshape(t_blk, B, 3 * H).astype(_BF16)
    gik_ref[...] = gik.reshape(t_blk, B, 3 * H).astype(_BF16)


def _gru_kernel(giq_ref, gik_ref, whh_ref, whhk_ref, bhh_ref, bhhk_ref,
                wh_ref, bh_ref, q_ref, k_ref, hq_ref, hk_ref, whhk_s,
                *, t_chunk, n_chunks):
    i = pl.program_id(0)

    @pl.when(i == 0)
    def _init():
        hq_ref[...] = jnp.zeros_like(hq_ref)
        hk_ref[...] = jnp.zeros_like(hk_ref)
        whhk_s[...] = whhk_ref[...].astype(_BF16)

    bhh = bhh_ref[...]
    bhhk = bhhk_ref[...]

    def _step(h, gi_t, w_ref, b):
        gh = jnp.dot(h.astype(_BF16), w_ref[...],
                     preferred_element_type=_F32) + b
        ir = gi_t[:, 0:H]
        iz = gi_t[:, H:2 * H]
        inn = gi_t[:, 2 * H:]
        hr = gh[:, 0:H]
        hz = gh[:, H:2 * H]
        hn = gh[:, 2 * H:]
        r = jax.nn.sigmoid(ir + hr)
        z = jax.nn.sigmoid(iz + hz)
        n = jnp.tanh(inn + r * hn)
        return (1.0 - z) * n + z * h

    hq = hq_ref[...]
    hk = hk_ref[...]
    for j in range(t_chunk):
        hq = _step(hq, giq_ref[j], whh_ref, bhh)
        hk = _step(hk, gik_ref[j], whhk_s, bhhk)
    hq_ref[...] = hq
    hk_ref[...] = hk

    @pl.when(i == n_chunks - 1)
    def _head():
        wh = wh_ref[...]
        bh = bh_ref[...]

        def _out(h):
            c = jnp.dot(jnp.maximum(h, 0.0).astype(_BF16), wh,
                        preferred_element_type=_F32) + bh
            nrm = jnp.sqrt(jnp.sum(c * c, axis=1, keepdims=True))
            return c / jnp.maximum(nrm, 1e-12)

        q_ref[...] = _out(hq)
        k_ref[...] = _out(hk)


def kernel(rgb_anchor, flow_anchor, labels, W1, b1, g1, be1, Wih, Whh, bih,
           bhh, W1k, b1k, g1k, be1k, Wihk, Whhk, bihk, bhhk, Wh, bh, queues,
           queue_ptrs):
    del labels
    rgb2d = rgb_anchor.reshape(B * T, DRGB).astype(_BF16)
    flow2d = flow_anchor.reshape(B * T, DFLOW).astype(_BF16)
    w1_bf = W1.astype(_BF16)
    w1k_bf = W1k.astype(_BF16)
    wih_bf = Wih.astype(_BF16)
    whh_bf = Whh.astype(_BF16)
    wh_bf = Wh.astype(_BF16)
    b1r = b1.reshape(1, EMB)
    b1kr = b1k.reshape(1, EMB)
    g1r = g1.reshape(1, EMB)
    be1r = be1.reshape(1, EMB)
    g1kr = g1k.reshape(1, EMB)
    be1kr = be1k.reshape(1, EMB)
    bihr = bih.reshape(1, 3 * H)
    bihkr = bihk.reshape(1, 3 * H)
    bhhr = bhh.reshape(1, 3 * H)
    bhhkr = bhhk.reshape(1, 3 * H)
    bhr = bh.reshape(1, CD)

    full = lambda shape: pl.BlockSpec(shape, lambda i: (0,) * len(shape))

    whhk_up = pl.kernel(
        _sc_mom_kernel,
        mesh=plsc.VectorSubcoreMesh(core_axis_name="c", subcore_axis_name="s"),
        out_type=jax.ShapeDtypeStruct((H, 3 * H), _F32),
        scratch_types=[
            pltpu.VMEM((8, 3 * H), _F32),
            pltpu.VMEM((8, 3 * H), _F32),
        ],
    )(Whh, Whhk)

    rows = B * T
    blk1 = 512
    n1 = rows // blk1
    nb1 = blk1 // T
    mblk = H // n1
    xq, xk, wihk_bf, bihk_c, bhhk_c = pl.pallas_call(
        functools.partial(_mm1_kernel, rows_per_blk=blk1),
        grid=(n1,),
        in_specs=[
            pl.BlockSpec((blk1, DRGB), lambda i: (i, 0)),
            pl.BlockSpec((blk1, DFLOW), lambda i: (i, 0)),
            full((DRGB + DFLOW, EMB)),
            full((DRGB + DFLOW, EMB)),
            full((1, EMB)), full((1, EMB)), full((1, EMB)),
            full((1, EMB)), full((1, EMB)), full((1, EMB)),
            pl.BlockSpec((mblk, 3 * H), lambda i: (i, 0)),
            pl.BlockSpec((mblk, 3 * H), lambda i: (i, 0)),
            full((1, 3 * H)), full((1, 3 * H)),
            full((1, 3 * H)), full((1, 3 * H)),
        ],
        out_specs=[
            pl.BlockSpec((T, nb1, EMB), lambda i: (0, i, 0)),
            pl.BlockSpec((T, nb1, EMB), lambda i: (0, i, 0)),
            pl.BlockSpec((mblk, 3 * H), lambda i: (i, 0)),
            full((1, 3 * H)),
            full((1, 3 * H)),
        ],
        out_shape=[
            jax.ShapeDtypeStruct((T, B, EMB), _BF16),
            jax.ShapeDtypeStruct((T, B, EMB), _BF16),
            jax.ShapeDtypeStruct((H, 3 * H), _BF16),
            jax.ShapeDtypeStruct((1, 3 * H), _F32),
            jax.ShapeDtypeStruct((1, 3 * H), _F32),
        ],
        compiler_params=pltpu.CompilerParams(
            dimension_semantics=("arbitrary",)),
    )(rgb2d, flow2d, w1_bf, w1k_bf, b1r, b1kr, g1r, be1r, g1kr, be1kr,
      Wih, Wihk, bihr, bihkr, bhhr, bhhkr)

    t_blk2 = 8
    n2 = T // t_blk2
    giq, gik = pl.pallas_call(
        functools.partial(_mm2_kernel, t_blk=t_blk2),
        grid=(n2,),
        in_specs=[
            pl.BlockSpec((t_blk2, B, EMB), lambda i: (i, 0, 0)),
            pl.BlockSpec((t_blk2, B, EMB), lambda i: (i, 0, 0)),
            full((EMB, 3 * H)),
            full((EMB, 3 * H)),
            full((1, 3 * H)), full((1, 3 * H)),
        ],
        out_specs=[
            pl.BlockSpec((t_blk2, B, 3 * H), lambda i: (i, 0, 0)),
            pl.BlockSpec((t_blk2, B, 3 * H), lambda i: (i, 0, 0)),
        ],
        out_shape=[
            jax.ShapeDtypeStruct((T, B, 3 * H), _BF16),
            jax.ShapeDtypeStruct((T, B, 3 * H), _BF16),
        ],
        compiler_params=pltpu.CompilerParams(
            dimension_semantics=("arbitrary",)),
    )(xq, xk, wih_bf, wihk_bf, bihr, bihk_c)

    t_chunk = 8
    n_chunks = T // t_chunk
    q_cls, k_cls = pl.pallas_call(
        functools.partial(_gru_kernel, t_chunk=t_chunk, n_chunks=n_chunks),
        grid=(n_chunks,),
        in_specs=[
            pl.BlockSpec((t_chunk, B, 3 * H), lambda i: (i, 0, 0)),
            pl.BlockSpec((t_chunk, B, 3 * H), lambda i: (i, 0, 0)),
            full((H, 3 * H)),
            full((H, 3 * H)),
            full((1, 3 * H)), full((1, 3 * H)),
            full((H, CD)), full((1, CD)),
        ],
        out_specs=[
            full((B, CD)),
            full((B, CD)),
        ],
        out_shape=[
            jax.ShapeDtypeStruct((B, CD), _F32),
            jax.ShapeDtypeStruct((B, CD), _F32),
        ],
        scratch_shapes=[
            pltpu.VMEM((B, H), _F32),
            pltpu.VMEM((B, H), _F32),
            pltpu.VMEM((H, 3 * H), _BF16),
        ],
        compiler_params=pltpu.CompilerParams(
            dimension_semantics=("arbitrary",)),
    )(giq, gik, whh_bf, whhk_up, bhhr, bhhk_c, wh_bf, bhr)

    return (q_cls, k_cls, queues, queue_ptrs)


# speedup vs baseline: 1.0552x; 1.0552x over previous
import functools

import jax
import jax.numpy as jnp
from jax.experimental import pallas as pl
from jax.experimental.pallas import tpu as pltpu

B = 64
T = 64
DRGB = 2048
DFLOW = 2048
EMB = 1024
H = 1024
CD = 128
MOM = 0.999

_F32 = jnp.float32
_BF16 = jnp.bfloat16


def _mm1_kernel(rgb_ref, flow_ref, w1_ref, w1k_ref, b1_ref, b1k_ref,
                g1_ref, be1_ref, g1k_ref, be1k_ref,
                wih_ref, wihk_ref, whh_ref, whhk_ref,
                bih_ref, bihk_ref, bhh_ref, bhhk_ref,
                xq_ref, xk_ref, oih_ref, ohh_ref, obih_ref, obhh_ref,
                *, rows_per_blk):
    oih_ref[...] = (MOM * wihk_ref[...]
                    + (1.0 - MOM) * wih_ref[...]).astype(_BF16)
    ohh_ref[...] = (MOM * whhk_ref[...]
                    + (1.0 - MOM) * whh_ref[...]).astype(_BF16)

    @pl.when(pl.program_id(0) == 0)
    def _biases():
        obih_ref[...] = MOM * bihk_ref[...] + (1.0 - MOM) * bih_ref[...]
        obhh_ref[...] = MOM * bhhk_ref[...] + (1.0 - MOM) * bhh_ref[...]

    rgb = rgb_ref[...]
    flow = flow_ref[...]
    yq = (jnp.dot(rgb, w1_ref[0:DRGB, :], preferred_element_type=_F32)
          + jnp.dot(flow, w1_ref[DRGB:, :], preferred_element_type=_F32)
          + b1_ref[...])
    ykraw = (jnp.dot(rgb, w1k_ref[0:DRGB, :], preferred_element_type=_F32)
             + jnp.dot(flow, w1k_ref[DRGB:, :], preferred_element_type=_F32)
             + b1k_ref[...])
    yk = MOM * ykraw + (1.0 - MOM) * yq
    nb = rows_per_blk // T

    def _ln_relu(y, g, be):
        mu = jnp.mean(y, axis=-1, keepdims=True)
        var = jnp.mean(y * y, axis=-1, keepdims=True) - mu * mu
        a = jax.lax.rsqrt(var + 1e-5) * g
        out = y * a + (be - mu * a)
        out = jnp.maximum(out, 0.0).astype(_BF16)
        return out.reshape(nb, T, EMB).transpose(1, 0, 2)

    g1 = g1_ref[...]
    be1 = be1_ref[...]
    g1k = MOM * g1k_ref[...] + (1.0 - MOM) * g1
    be1k = MOM * be1k_ref[...] + (1.0 - MOM) * be1
    xq_ref[...] = _ln_relu(yq, g1, be1)
    xk_ref[...] = _ln_relu(yk, g1k, be1k)


def _mm2_kernel(xq_ref, xk_ref, wih_ref, wihk_ref, bih_ref, bihk_ref,
                giq_ref, gik_ref, *, t_blk):
    bih = bih_ref[...]
    bihk = bihk_ref[...]
    xq = xq_ref[...].reshape(t_blk * B, EMB)
    xk = xk_ref[...].reshape(t_blk * B, EMB)
    giq = jnp.dot(xq, wih_ref[...], preferred_element_type=_F32) + bih
    gik = jnp.dot(xk, wihk_ref[...], preferred_element_type=_F32) + bihk
    giq_ref[...] = giq.reshape(t_blk, B, 3 * H).astype(_BF16)
    gik_ref[...] = gik.reshape(t_blk, B, 3 * H).astype(_BF16)


def _gru_kernel(giq_ref, gik_ref, whh_ref, whhk_ref, bhh_ref, bhhk_ref,
                wh_ref, bh_ref, q_ref, k_ref, hq_ref, hk_ref,
                gq_ref, gk_ref, *, t_chunk, n_chunks, n_kc):
    i = pl.program_id(0)
    cs = H // n_kc

    @pl.when(i == 0)
    def _init():
        hq_ref[...] = jnp.zeros_like(hq_ref)
        hk_ref[...] = jnp.zeros_like(hk_ref)
        gq_ref[...] = jnp.zeros_like(gq_ref)
        gk_ref[...] = jnp.zeros_like(gk_ref)

    bhh = bhh_ref[...]
    bhhk = bhhk_ref[...]

    def _step(h_chunks, gh, gi_t, w_ref, b):
        new_chunks = []
        parts = []
        for c in range(n_kc):
            lo = c * cs
            hi = lo + cs
            r = jax.nn.sigmoid(gi_t[:, lo:hi] + gh[:, lo:hi] + b[:, lo:hi])
            z = jax.nn.sigmoid(gi_t[:, H + lo:H + hi] + gh[:, H + lo:H + hi]
                               + b[:, H + lo:H + hi])
            n = jnp.tanh(gi_t[:, 2 * H + lo:2 * H + hi]
                         + r * (gh[:, 2 * H + lo:2 * H + hi]
                                + b[:, 2 * H + lo:2 * H + hi]))
            h_c = (1.0 - z) * n + z * h_chunks[c]
            new_chunks.append(h_c)
            parts.append(jnp.dot(h_c.astype(_BF16), w_ref[lo:hi, :],
                                 preferred_element_type=_F32))
        gh_next = parts[0]
        for p in parts[1:]:
            gh_next = gh_next + p
        return new_chunks, gh_next

    hq_chunks = [hq_ref[:, c * cs:(c + 1) * cs] for c in range(n_kc)]
    hk_chunks = [hk_ref[:, c * cs:(c + 1) * cs] for c in range(n_kc)]
    ghq = gq_ref[...]
    ghk = gk_ref[...]
    for j in range(t_chunk):
        hq_chunks, ghq = _step(hq_chunks, ghq, giq_ref[j], whh_ref, bhh)
        hk_chunks, ghk = _step(hk_chunks, ghk, gik_ref[j], whhk_ref, bhhk)
    for c in range(n_kc):
        hq_ref[:, c * cs:(c + 1) * cs] = hq_chunks[c]
        hk_ref[:, c * cs:(c + 1) * cs] = hk_chunks[c]
    gq_ref[...] = ghq
    gk_ref[...] = ghk

    @pl.when(i == n_chunks - 1)
    def _head():
        wh = wh_ref[...]
        bh = bh_ref[...]

        def _out(chunks):
            h = jnp.concatenate(chunks, axis=1)
            c = jnp.dot(jnp.maximum(h, 0.0).astype(_BF16), wh,
                        preferred_element_type=_F32) + bh
            nrm = jnp.sqrt(jnp.sum(c * c, axis=1, keepdims=True))
            return c / jnp.maximum(nrm, 1e-12)

        q_ref[...] = _out(hq_chunks)
        k_ref[...] = _out(hk_chunks)


def kernel(rgb_anchor, flow_anchor, labels, W1, b1, g1, be1, Wih, Whh, bih,
           bhh, W1k, b1k, g1k, be1k, Wihk, Whhk, bihk, bhhk, Wh, bh, queues,
           queue_ptrs):
    del labels
    rgb2d = rgb_anchor.reshape(B * T, DRGB).astype(_BF16)
    flow2d = flow_anchor.reshape(B * T, DFLOW).astype(_BF16)
    w1_bf = W1.astype(_BF16)
    w1k_bf = W1k.astype(_BF16)
    wih_bf = Wih.astype(_BF16)
    whh_bf = Whh.astype(_BF16)
    wh_bf = Wh.astype(_BF16)
    b1r = b1.reshape(1, EMB)
    b1kr = b1k.reshape(1, EMB)
    g1r = g1.reshape(1, EMB)
    be1r = be1.reshape(1, EMB)
    g1kr = g1k.reshape(1, EMB)
    be1kr = be1k.reshape(1, EMB)
    bihr = bih.reshape(1, 3 * H)
    bihkr = bihk.reshape(1, 3 * H)
    bhhr = bhh.reshape(1, 3 * H)
    bhhkr = bhhk.reshape(1, 3 * H)
    bhr = bh.reshape(1, CD)

    full = lambda shape: pl.BlockSpec(shape, lambda i: (0,) * len(shape))

    rows = B * T
    blk1 = 512
    n1 = rows // blk1
    nb1 = blk1 // T
    mblk = H // n1
    xq, xk, wihk_bf, whhk_bf, bihk_c, bhhk_c = pl.pallas_call(
        functools.partial(_mm1_kernel, rows_per_blk=blk1),
        grid=(n1,),
        in_specs=[
            pl.BlockSpec((blk1, DRGB), lambda i: (i, 0)),
            pl.BlockSpec((blk1, DFLOW), lambda i: (i, 0)),
            full((DRGB + DFLOW, EMB)),
            full((DRGB + DFLOW, EMB)),
            full((1, EMB)), full((1, EMB)), full((1, EMB)),
            full((1, EMB)), full((1, EMB)), full((1, EMB)),
            pl.BlockSpec((mblk, 3 * H), lambda i: (i, 0)),
            pl.BlockSpec((mblk, 3 * H), lambda i: (i, 0)),
            pl.BlockSpec((mblk, 3 * H), lambda i: (i, 0)),
            pl.BlockSpec((mblk, 3 * H), lambda i: (i, 0)),
            full((1, 3 * H)), full((1, 3 * H)),
            full((1, 3 * H)), full((1, 3 * H)),
        ],
        out_specs=[
            pl.BlockSpec((T, nb1, EMB), lambda i: (0, i, 0)),
            pl.BlockSpec((T, nb1, EMB), lambda i: (0, i, 0)),
            pl.BlockSpec((mblk, 3 * H), lambda i: (i, 0)),
            pl.BlockSpec((mblk, 3 * H), lambda i: (i, 0)),
            full((1, 3 * H)),
            full((1, 3 * H)),
        ],
        out_shape=[
            jax.ShapeDtypeStruct((T, B, EMB), _BF16),
            jax.ShapeDtypeStruct((T, B, EMB), _BF16),
            jax.ShapeDtypeStruct((H, 3 * H), _BF16),
            jax.ShapeDtypeStruct((H, 3 * H), _BF16),
            jax.ShapeDtypeStruct((1, 3 * H), _F32),
            jax.ShapeDtypeStruct((1, 3 * H), _F32),
        ],
        compiler_params=pltpu.CompilerParams(
            dimension_semantics=("arbitrary",)),
    )(rgb2d, flow2d, w1_bf, w1k_bf, b1r, b1kr, g1r, be1r, g1kr, be1kr,
      Wih, Wihk, Whh, Whhk, bihr, bihkr, bhhr, bhhkr)

    t_blk2 = 8
    n2 = T // t_blk2
    giq, gik = pl.pallas_call(
        functools.partial(_mm2_kernel, t_blk=t_blk2),
        grid=(n2,),
        in_specs=[
            pl.BlockSpec((t_blk2, B, EMB), lambda i: (i, 0, 0)),
            pl.BlockSpec((t_blk2, B, EMB), lambda i: (i, 0, 0)),
            full((EMB, 3 * H)),
            full((EMB, 3 * H)),
            full((1, 3 * H)), full((1, 3 * H)),
        ],
        out_specs=[
            pl.BlockSpec((t_blk2, B, 3 * H), lambda i: (i, 0, 0)),
            pl.BlockSpec((t_blk2, B, 3 * H), lambda i: (i, 0, 0)),
        ],
        out_shape=[
            jax.ShapeDtypeStruct((T, B, 3 * H), _BF16),
            jax.ShapeDtypeStruct((T, B, 3 * H), _BF16),
        ],
        compiler_params=pltpu.CompilerParams(
            dimension_semantics=("arbitrary",)),
    )(xq, xk, wih_bf, wihk_bf, bihr, bihk_c)

    t_chunk = 8
    n_chunks = T // t_chunk
    q_cls, k_cls = pl.pallas_call(
        functools.partial(_gru_kernel, t_chunk=t_chunk, n_chunks=n_chunks,
                          n_kc=4),
        grid=(n_chunks,),
        in_specs=[
            pl.BlockSpec((t_chunk, B, 3 * H), lambda i: (i, 0, 0)),
            pl.BlockSpec((t_chunk, B, 3 * H), lambda i: (i, 0, 0)),
            full((H, 3 * H)),
            full((H, 3 * H)),
            full((1, 3 * H)), full((1, 3 * H)),
            full((H, CD)), full((1, CD)),
        ],
        out_specs=[
            full((B, CD)),
            full((B, CD)),
        ],
        out_shape=[
            jax.ShapeDtypeStruct((B, CD), _F32),
            jax.ShapeDtypeStruct((B, CD), _F32),
        ],
        scratch_shapes=[
            pltpu.VMEM((B, H), _F32),
            pltpu.VMEM((B, H), _F32),
            pltpu.VMEM((B, 3 * H), _F32),
            pltpu.VMEM((B, 3 * H), _F32),
        ],
        compiler_params=pltpu.CompilerParams(
            dimension_semantics=("arbitrary",)),
    )(giq, gik, whh_bf, whhk_bf, bhhr, bhhk_c, wh_bf, bhr)

    return (q_cls, k_cls, queues, queue_ptrs)


# speedup vs baseline: 1.0761x; 1.0198x over previous
import functools

import jax
import jax.numpy as jnp
from jax.experimental import pallas as pl
from jax.experimental.pallas import tpu as pltpu

B = 64
T = 64
DRGB = 2048
DFLOW = 2048
EMB = 1024
H = 1024
CD = 128
MOM = 0.999

_F32 = jnp.float32
_BF16 = jnp.bfloat16


def _mm1_kernel(rgb_ref, flow_ref, w1_ref, w1k_ref, b1_ref, b1k_ref,
                g1_ref, be1_ref, g1k_ref, be1k_ref,
                wih_ref, wihk_ref, whh_ref, whhk_ref,
                bih_ref, bihk_ref, bhh_ref, bhhk_ref,
                xq_ref, xk_ref, oih_ref, ohh_ref, obih_ref, obhh_ref,
                *, rows_per_blk):
    oih_ref[...] = (MOM * wihk_ref[...]
                    + (1.0 - MOM) * wih_ref[...]).astype(_BF16)
    ohh_ref[...] = (MOM * whhk_ref[...]
                    + (1.0 - MOM) * whh_ref[...]).astype(_BF16)

    @pl.when(pl.program_id(0) == 0)
    def _biases():
        obih_ref[...] = MOM * bihk_ref[...] + (1.0 - MOM) * bih_ref[...]
        obhh_ref[...] = MOM * bhhk_ref[...] + (1.0 - MOM) * bhh_ref[...]

    rgb = rgb_ref[...]
    flow = flow_ref[...]
    yq = (jnp.dot(rgb, w1_ref[0:DRGB, :], preferred_element_type=_F32)
          + jnp.dot(flow, w1_ref[DRGB:, :], preferred_element_type=_F32)
          + b1_ref[...])
    ykraw = (jnp.dot(rgb, w1k_ref[0:DRGB, :], preferred_element_type=_F32)
             + jnp.dot(flow, w1k_ref[DRGB:, :], preferred_element_type=_F32)
             + b1k_ref[...])
    yk = MOM * ykraw + (1.0 - MOM) * yq
    nb = rows_per_blk // T

    def _ln_relu(y, g, be):
        mu = jnp.mean(y, axis=-1, keepdims=True)
        var = jnp.mean(y * y, axis=-1, keepdims=True) - mu * mu
        a = jax.lax.rsqrt(var + 1e-5) * g
        out = y * a + (be - mu * a)
        out = jnp.maximum(out, 0.0).astype(_BF16)
        return out.reshape(nb, T, EMB).transpose(1, 0, 2)

    g1 = g1_ref[...]
    be1 = be1_ref[...]
    g1k = MOM * g1k_ref[...] + (1.0 - MOM) * g1
    be1k = MOM * be1k_ref[...] + (1.0 - MOM) * be1
    xq_ref[...] = _ln_relu(yq, g1, be1)
    xk_ref[...] = _ln_relu(yk, g1k, be1k)


def _gru_kernel(xq_ref, xk_ref, wih_ref, wihk_ref, bih_ref, bihk_ref,
                whh_ref, whhk_ref, bhh_ref, bhhk_ref,
                wh_ref, bh_ref, q_ref, k_ref, hq_ref, hk_ref,
                gq_ref, gk_ref, *, t_chunk, n_chunks, n_kc):
    i = pl.program_id(0)
    cs = H // n_kc

    @pl.when(i == 0)
    def _init():
        hq_ref[...] = jnp.zeros_like(hq_ref)
        hk_ref[...] = jnp.zeros_like(hk_ref)
        gq_ref[...] = jnp.zeros_like(gq_ref)
        gk_ref[...] = jnp.zeros_like(gk_ref)

    bhh = bhh_ref[...]
    bhhk = bhhk_ref[...]

    giq = (jnp.dot(xq_ref[...].reshape(t_chunk * B, EMB), wih_ref[...],
                   preferred_element_type=_F32)
           + bih_ref[...]).reshape(t_chunk, B, 3 * H)
    gik = (jnp.dot(xk_ref[...].reshape(t_chunk * B, EMB), wihk_ref[...],
                   preferred_element_type=_F32)
           + bihk_ref[...]).reshape(t_chunk, B, 3 * H)

    def _step(h_chunks, gh, gi_t, w_ref, b):
        new_chunks = []
        parts = []
        for c in range(n_kc):
            lo = c * cs
            hi = lo + cs
            r = jax.nn.sigmoid(gi_t[:, lo:hi] + gh[:, lo:hi] + b[:, lo:hi])
            z = jax.nn.sigmoid(gi_t[:, H + lo:H + hi] + gh[:, H + lo:H + hi]
                               + b[:, H + lo:H + hi])
            n = jnp.tanh(gi_t[:, 2 * H + lo:2 * H + hi]
                         + r * (gh[:, 2 * H + lo:2 * H + hi]
                                + b[:, 2 * H + lo:2 * H + hi]))
            h_c = (1.0 - z) * n + z * h_chunks[c]
            new_chunks.append(h_c)
            parts.append(jnp.dot(h_c.astype(_BF16), w_ref[lo:hi, :],
                                 preferred_element_type=_F32))
        gh_next = parts[0]
        for p in parts[1:]:
            gh_next = gh_next + p
        return new_chunks, gh_next

    hq_chunks = [hq_ref[:, c * cs:(c + 1) * cs] for c in range(n_kc)]
    hk_chunks = [hk_ref[:, c * cs:(c + 1) * cs] for c in range(n_kc)]
    ghq = gq_ref[...]
    ghk = gk_ref[...]
    for j in range(t_chunk):
        hq_chunks, ghq = _step(hq_chunks, ghq, giq[j], whh_ref, bhh)
        hk_chunks, ghk = _step(hk_chunks, ghk, gik[j], whhk_ref, bhhk)
    for c in range(n_kc):
        hq_ref[:, c * cs:(c + 1) * cs] = hq_chunks[c]
        hk_ref[:, c * cs:(c + 1) * cs] = hk_chunks[c]
    gq_ref[...] = ghq
    gk_ref[...] = ghk

    @pl.when(i == n_chunks - 1)
    def _head():
        wh = wh_ref[...]
        bh = bh_ref[...]

        def _out(chunks):
            h = jnp.concatenate(chunks, axis=1)
            c = jnp.dot(jnp.maximum(h, 0.0).astype(_BF16), wh,
                        preferred_element_type=_F32) + bh
            nrm = jnp.sqrt(jnp.sum(c * c, axis=1, keepdims=True))
            return c / jnp.maximum(nrm, 1e-12)

        q_ref[...] = _out(hq_chunks)
        k_ref[...] = _out(hk_chunks)


def kernel(rgb_anchor, flow_anchor, labels, W1, b1, g1, be1, Wih, Whh, bih,
           bhh, W1k, b1k, g1k, be1k, Wihk, Whhk, bihk, bhhk, Wh, bh, queues,
           queue_ptrs):
    del labels
    rgb2d = rgb_anchor.reshape(B * T, DRGB).astype(_BF16)
    flow2d = flow_anchor.reshape(B * T, DFLOW).astype(_BF16)
    w1_bf = W1.astype(_BF16)
    w1k_bf = W1k.astype(_BF16)
    wih_bf = Wih.astype(_BF16)
    whh_bf = Whh.astype(_BF16)
    wh_bf = Wh.astype(_BF16)
    b1r = b1.reshape(1, EMB)
    b1kr = b1k.reshape(1, EMB)
    g1r = g1.reshape(1, EMB)
    be1r = be1.reshape(1, EMB)
    g1kr = g1k.reshape(1, EMB)
    be1kr = be1k.reshape(1, EMB)
    bihr = bih.reshape(1, 3 * H)
    bihkr = bihk.reshape(1, 3 * H)
    bhhr = bhh.reshape(1, 3 * H)
    bhhkr = bhhk.reshape(1, 3 * H)
    bhr = bh.reshape(1, CD)

    full = lambda shape: pl.BlockSpec(shape, lambda i: (0,) * len(shape))

    rows = B * T
    blk1 = 512
    n1 = rows // blk1
    nb1 = blk1 // T
    mblk = H // n1
    xq, xk, wihk_bf, whhk_bf, bihk_c, bhhk_c = pl.pallas_call(
        functools.partial(_mm1_kernel, rows_per_blk=blk1),
        grid=(n1,),
        in_specs=[
            pl.BlockSpec((blk1, DRGB), lambda i: (i, 0)),
            pl.BlockSpec((blk1, DFLOW), lambda i: (i, 0)),
            full((DRGB + DFLOW, EMB)),
            full((DRGB + DFLOW, EMB)),
            full((1, EMB)), full((1, EMB)), full((1, EMB)),
            full((1, EMB)), full((1, EMB)), full((1, EMB)),
            pl.BlockSpec((mblk, 3 * H), lambda i: (i, 0)),
            pl.BlockSpec((mblk, 3 * H), lambda i: (i, 0)),
            pl.BlockSpec((mblk, 3 * H), lambda i: (i, 0)),
            pl.BlockSpec((mblk, 3 * H), lambda i: (i, 0)),
            full((1, 3 * H)), full((1, 3 * H)),
            full((1, 3 * H)), full((1, 3 * H)),
        ],
        out_specs=[
            pl.BlockSpec((T, nb1, EMB), lambda i: (0, i, 0)),
            pl.BlockSpec((T, nb1, EMB), lambda i: (0, i, 0)),
            pl.BlockSpec((mblk, 3 * H), lambda i: (i, 0)),
            pl.BlockSpec((mblk, 3 * H), lambda i: (i, 0)),
            full((1, 3 * H)),
            full((1, 3 * H)),
        ],
        out_shape=[
            jax.ShapeDtypeStruct((T, B, EMB), _BF16),
            jax.ShapeDtypeStruct((T, B, EMB), _BF16),
            jax.ShapeDtypeStruct((H, 3 * H), _BF16),
            jax.ShapeDtypeStruct((H, 3 * H), _BF16),
            jax.ShapeDtypeStruct((1, 3 * H), _F32),
            jax.ShapeDtypeStruct((1, 3 * H), _F32),
        ],
        compiler_params=pltpu.CompilerParams(
            dimension_semantics=("arbitrary",)),
    )(rgb2d, flow2d, w1_bf, w1k_bf, b1r, b1kr, g1r, be1r, g1kr, be1kr,
      Wih, Wihk, Whh, Whhk, bihr, bihkr, bhhr, bhhkr)

    t_chunk = 8
    n_chunks = T // t_chunk
    q_cls, k_cls = pl.pallas_call(
        functools.partial(_gru_kernel, t_chunk=t_chunk, n_chunks=n_chunks,
                          n_kc=4),
        grid=(n_chunks,),
        in_specs=[
            pl.BlockSpec((t_chunk, B, EMB), lambda i: (i, 0, 0)),
            pl.BlockSpec((t_chunk, B, EMB), lambda i: (i, 0, 0)),
            full((EMB, 3 * H)),
            full((EMB, 3 * H)),
            full((1, 3 * H)), full((1, 3 * H)),
            full((H, 3 * H)),
            full((H, 3 * H)),
            full((1, 3 * H)), full((1, 3 * H)),
            full((H, CD)), full((1, CD)),
        ],
        out_specs=[
            full((B, CD)),
            full((B, CD)),
        ],
        out_shape=[
            jax.ShapeDtypeStruct((B, CD), _F32),
            jax.ShapeDtypeStruct((B, CD), _F32),
        ],
        scratch_shapes=[
            pltpu.VMEM((B, H), _F32),
            pltpu.VMEM((B, H), _F32),
            pltpu.VMEM((B, 3 * H), _F32),
            pltpu.VMEM((B, 3 * H), _F32),
        ],
        compiler_params=pltpu.CompilerParams(
            dimension_semantics=("arbitrary",)),
    )(xq, xk, wih_bf, wihk_bf, bihr, bihk_c,
      whh_bf, whhk_bf, bhhr, bhhk_c, wh_bf, bhr)

    return (q_cls, k_cls, queues, queue_ptrs)
